# Initial kernel scaffold; baseline (speedup 1.0000x reference)
import jax
import jax.numpy as jnp
from jax.experimental import pallas as pl


def kernel(weight):
    raise NotImplementedError("write your pallas kernel here")


# speedup vs baseline: 11.9310x; 11.9310x over previous
import functools
import math

import jax
import jax.numpy as jnp
from jax import lax
from jax.experimental import pallas as pl
from jax.experimental.pallas import tpu as pltpu
from jax.experimental.pallas import tpu_sc as plsc

_SPARSITY = 0.9
_TEMP = 1e-05

_N = 4096 * 4096
_LIM = max(0, int((1.0 - _SPARSITY) * _N) - 1)
_R_H = _LIM + 1
_R_T = _LIM + 2

_NTILES = 16
_PER_TILE = _N // _NTILES
_CH = 8192
_NCH = _PER_TILE // _CH

_HSIZE = 32768
_HALF = _HSIZE // 2


def _sc_select(w_hbm, out_hbm, tbl_h, tbl_t, res, buf0, buf1, hist, seg,
               tbuf, rbuf, tmp16, outb, sem0, sem1, sem_seg, sa_all):
    wid = lax.axis_index("s")
    lane = lax.iota(jnp.int32, 16)
    ones = jnp.ones((16,), jnp.int32)
    zeros16 = jnp.zeros((16,), jnp.int32)
    base0 = wid * _PER_TILE

    def zero_hist():
        def body(i, _):
            hist[pl.ds(i * 16, 16)] = zeros16
            return 0
        lax.fori_loop(0, _HSIZE // 16, body, 0)

    def stream(process):
        pltpu.async_copy(w_hbm.at[pl.ds(base0, _CH)], buf0, sem0)

        def pair(j, _):
            c0 = 2 * j
            pltpu.async_copy(
                w_hbm.at[pl.ds(base0 + (c0 + 1) * _CH, _CH)], buf1, sem1)
            pltpu.make_async_copy(
                w_hbm.at[pl.ds(0, _CH)], buf0, sem0).wait()
            process(buf0)

            @pl.when(j < _NCH // 2 - 1)
            def _():
                pltpu.async_copy(
                    w_hbm.at[pl.ds(base0 + (c0 + 2) * _CH, _CH)], buf0, sem0)

            pltpu.make_async_copy(
                w_hbm.at[pl.ds(0, _CH)], buf1, sem1).wait()
            process(buf1)
            return 0

        lax.fori_loop(0, _NCH // 2, pair, 0)

    def extract(vec, pos):
        return jnp.sum(jnp.where(lane == pos, vec, 0))

    def find_chain(base, tbl, rank, seg_entries, ndig_seg, chain_row):
        copies = []
        for t in range(_NTILES):
            copies.append(pltpu.async_copy(
                sa_all.at[t, pl.ds(base + wid * seg_entries, seg_entries)],
                seg.at[t, pl.ds(0, seg_entries)], sem_seg))
        for c in copies:
            c.wait()

        def tot_body(k, acc):
            for t in range(_NTILES):
                acc = acc + seg[t, pl.ds(k * 16, 16)]
            return acc
        accv = lax.fori_loop(0, seg_entries // 16, tot_body, zeros16)
        seg_sum = jnp.sum(accv)

        tmp16[...] = jnp.broadcast_to(seg_sum, (16,)).astype(jnp.int32)
        pltpu.sync_copy(tmp16, tbl.at[wid])
        plsc.subcore_barrier()
        pltpu.sync_copy(tbl, tbuf)
        suf = jnp.int32(0)
        for t in range(_NTILES):
            s_t = tbuf[t, pl.ds(0, 16)][0]
            suf = suf + jnp.where(jnp.int32(t) > wid, s_t, jnp.int32(0))

        def scan_body(i, carry):
            cum, found, d_sel, a_sel = carry
            k = ndig_seg - 1 - i
            dv = zeros16
            for t in range(_NTILES):
                dv = dv + seg[t, pl.ds(k * 16, 16)]
            tot = jnp.sum(dv)
            new_cum = cum + tot
            crossed = jnp.logical_and(suf + new_cum >= rank, found == 0)
            d_sel = jnp.where(crossed, k, d_sel)
            a_sel = jnp.where(crossed, suf + cum, a_sel)
            found = jnp.where(crossed, 1, found)
            return (new_cum, found, d_sel, a_sel)

        init = (jnp.int32(0), jnp.int32(0), jnp.int32(0), jnp.int32(0))
        _, _, d_loc, a_glob = lax.fori_loop(0, ndig_seg, scan_body, init)
        d_glob = wid * ndig_seg + d_loc

        mine = jnp.logical_and(suf < rank, rank <= suf + seg_sum)
        resv = jnp.where(lane == 0, d_glob,
                         jnp.where(lane == 1, a_glob, 0)).astype(jnp.int32)
        tmp16[...] = resv

        @pl.when(mine)
        def _():
            pltpu.sync_copy(tmp16, res.at[chain_row])

    p_h = jnp.int32(0)
    p_t = jnp.int32(0)
    r_h = jnp.int32(_R_H)
    r_t = jnp.int32(_R_T)

    for p in range(3):
        width = (11, 10, 10)[p]

        zero_hist()

        if p == 0:
            def process(buf):
                def body(i, _):
                    bits = buf[pl.ds(i * 16, 16)] & 0x7FFFFFFF
                    idx = (lax.shift_right_logical(bits, 16) & 0x7FF0) | lane
                    plsc.addupdate_scatter(hist, [idx], ones)
                    return 0
                lax.fori_loop(0, _CH // 16, body, 0)
        elif p == 1:
            ph, pt = p_h, p_t

            def process(buf):
                def body(i, _):
                    bits = buf[pl.ds(i * 16, 16)] & 0x7FFFFFFF
                    pref = lax.shift_right_logical(bits, 20)
                    idx = (lax.shift_right_logical(bits, 6) & 0x3FF0) | lane
                    plsc.addupdate_scatter(hist, [idx], ones,
                                           mask=pref == ph)
                    plsc.addupdate_scatter(hist, [idx | _HALF], ones,
                                           mask=pref == pt)
                    return 0
                lax.fori_loop(0, _CH // 16, body, 0)
        else:
            ph, pt = p_h, p_t

            def process(buf):
                def body(i, _):
                    bits = buf[pl.ds(i * 16, 16)] & 0x7FFFFFFF
                    pref = lax.shift_right_logical(bits, 10)
                    idx = (lax.shift_left(bits, 4) & 0x3FF0) | lane
                    plsc.addupdate_scatter(hist, [idx], ones,
                                           mask=pref == ph)
                    plsc.addupdate_scatter(hist, [idx | _HALF], ones,
                                           mask=pref == pt)
                    return 0
                lax.fori_loop(0, _CH // 16, body, 0)

        stream(process)

        pltpu.sync_copy(hist, sa_all.at[wid])
        plsc.subcore_barrier()

        if p == 0:
            find_chain(0, tbl_h, r_h, 2048, 128, 0)
            find_chain(0, tbl_t, r_t, 2048, 128, 1)
        else:
            find_chain(0, tbl_h, r_h, 1024, 64, 0)
            find_chain(_HALF, tbl_t, r_t, 1024, 64, 1)
        plsc.subcore_barrier()

        pltpu.sync_copy(res, rbuf)
        vh = rbuf[0, pl.ds(0, 16)]
        vt = rbuf[1, pl.ds(0, 16)]
        d_h, a_h = vh[0], vh[1]
        d_t, a_t = vt[0], vt[1]
        p_h = lax.shift_left(p_h, width) | d_h
        p_t = lax.shift_left(p_t, width) | d_t
        r_h = r_h - a_h
        r_t = r_t - a_t

    vi = jnp.where(lane == 0, p_h,
                   jnp.where(lane == 1, p_t, 0)).astype(jnp.int32)
    outb[...] = plsc.bitcast(vi, jnp.float32)

    @pl.when(wid == 0)
    def _():
        pltpu.sync_copy(outb, out_hbm)


@functools.partial(
    pl.kernel,
    out_type=(jax.ShapeDtypeStruct((16,), jnp.float32),
              jax.ShapeDtypeStruct((16, 16), jnp.int32),
              jax.ShapeDtypeStruct((16, 16), jnp.int32),
              jax.ShapeDtypeStruct((2, 16), jnp.int32)),
    mesh=plsc.VectorSubcoreMesh(core_axis_name="c", subcore_axis_name="s",
                                num_cores=1),
    compiler_params=pltpu.CompilerParams(needs_layout_passes=False),
    scratch_types=[
        pltpu.VMEM((_CH,), jnp.int32),
        pltpu.VMEM((_CH,), jnp.int32),
        pltpu.VMEM((_HSIZE,), jnp.int32),
        pltpu.VMEM((16, 2048), jnp.int32),
        pltpu.VMEM((16, 16), jnp.int32),
        pltpu.VMEM((2, 16), jnp.int32),
        pltpu.VMEM((16,), jnp.int32),
        pltpu.VMEM((16,), jnp.float32),
        pltpu.SemaphoreType.DMA,
        pltpu.SemaphoreType.DMA,
        pltpu.SemaphoreType.DMA,
        pltpu.VMEM_SHARED((16, _HSIZE), jnp.int32),
    ],
)
def _select_kernel(w_hbm, out_hbm, tbl_h, tbl_t, res, *rest):
    _sc_select(w_hbm, out_hbm, tbl_h, tbl_t, res, *rest)


_ROWS_PER_BLK = 256
_SQRT_INV2T = math.sqrt(0.5 / _TEMP)


def _apply_body(sel_ref, w_ref, o_ref):
    wh = sel_ref[0]
    wt = sel_ref[1]
    t = 0.5 * (wh + wt)
    a = (t * t) * (0.5 / _TEMP)
    w = w_ref[...]
    u = w * _SQRT_INV2T
    m = 0.5 + 0.5 * jnp.tanh(u * u - a)
    o_ref[...] = m * w


def kernel(weight):
    w_flat = weight.reshape(-1)
    w_i32 = lax.bitcast_convert_type(w_flat, jnp.int32)
    sel, _, _, _ = _select_kernel(w_i32)

    out = pl.pallas_call(
        _apply_body,
        grid=(weight.shape[0] // _ROWS_PER_BLK,),
        in_specs=[
            pl.BlockSpec(memory_space=pltpu.SMEM),
            pl.BlockSpec((_ROWS_PER_BLK, weight.shape[1]),
                         lambda i: (i, 0)),
        ],
        out_specs=pl.BlockSpec((_ROWS_PER_BLK, weight.shape[1]),
                               lambda i: (i, 0)),
        out_shape=jax.ShapeDtypeStruct(weight.shape, weight.dtype),
    )(sel, weight)
    return out


# speedup vs baseline: 41.3719x; 3.4676x over previous
import functools
import math

import jax
import jax.numpy as jnp
from jax import lax
from jax.experimental import pallas as pl
from jax.experimental.pallas import tpu as pltpu
from jax.experimental.pallas import tpu_sc as plsc

_SPARSITY = 0.9
_TEMP = 1e-05

_N = 4096 * 4096
_LIM = max(0, int((1.0 - _SPARSITY) * _N) - 1)
_R_H = _LIM + 1
_R_T = _LIM + 2

_NTILES = 16
_PER_TILE = _N // _NTILES
_CH = 8192
_NCH = _PER_TILE // _CH

_HSIZE = 32768
_HALF = _HSIZE // 2


def _sc_select(w_hbm, out_hbm, tbl_h, tbl_t, res, buf0, buf1, hist, seg,
               tbuf, rbuf, tmp16, outb, sem0, sem1, sem_seg, sa_all):
    wid = lax.axis_index("s")
    lane = lax.iota(jnp.int32, 16)
    ones = jnp.ones((16,), jnp.int32)
    zeros16 = jnp.zeros((16,), jnp.int32)
    base0 = wid * _PER_TILE

    def zero_hist():
        @plsc.parallel_loop(0, _HSIZE // 16, unroll=8)
        def _(i):
            hist[pl.ds(i * 16, 16)] = zeros16

    def stream(process):
        pltpu.async_copy(w_hbm.at[pl.ds(base0, _CH)], buf0, sem0)

        def pair(j, _):
            c0 = 2 * j
            pltpu.async_copy(
                w_hbm.at[pl.ds(base0 + (c0 + 1) * _CH, _CH)], buf1, sem1)
            pltpu.make_async_copy(
                w_hbm.at[pl.ds(0, _CH)], buf0, sem0).wait()
            process(buf0)

            @pl.when(j < _NCH // 2 - 1)
            def _():
                pltpu.async_copy(
                    w_hbm.at[pl.ds(base0 + (c0 + 2) * _CH, _CH)], buf0, sem0)

            pltpu.make_async_copy(
                w_hbm.at[pl.ds(0, _CH)], buf1, sem1).wait()
            process(buf1)
            return 0

        lax.fori_loop(0, _NCH // 2, pair, 0)

    def extract(vec, pos):
        return jnp.sum(jnp.where(lane == pos, vec, 0))

    def find_chain(base, tbl, rank, seg_entries, ndig_seg, chain_row):
        copies = []
        for t in range(_NTILES):
            copies.append(pltpu.async_copy(
                sa_all.at[t, pl.ds(base + wid * seg_entries, seg_entries)],
                seg.at[t, pl.ds(0, seg_entries)], sem_seg))
        for c in copies:
            c.wait()

        def tot_body(k, acc):
            for t in range(_NTILES):
                acc = acc + seg[t, pl.ds(k * 16, 16)]
            return acc
        accv = lax.fori_loop(0, seg_entries // 16, tot_body, zeros16)
        seg_sum = jnp.sum(accv)

        tmp16[...] = jnp.broadcast_to(seg_sum, (16,)).astype(jnp.int32)
        pltpu.sync_copy(tmp16, tbl.at[wid])
        plsc.subcore_barrier()
        pltpu.sync_copy(tbl, tbuf)
        suf = jnp.int32(0)
        for t in range(_NTILES):
            s_t = tbuf[t, pl.ds(0, 16)][0]
            suf = suf + jnp.where(jnp.int32(t) > wid, s_t, jnp.int32(0))

        def scan_body(i, carry):
            cum, found, d_sel, a_sel = carry
            k = ndig_seg - 1 - i
            dv = zeros16
            for t in range(_NTILES):
                dv = dv + seg[t, pl.ds(k * 16, 16)]
            tot = jnp.sum(dv)
            new_cum = cum + tot
            crossed = jnp.logical_and(suf + new_cum >= rank, found == 0)
            d_sel = jnp.where(crossed, k, d_sel)
            a_sel = jnp.where(crossed, suf + cum, a_sel)
            found = jnp.where(crossed, 1, found)
            return (new_cum, found, d_sel, a_sel)

        init = (jnp.int32(0), jnp.int32(0), jnp.int32(0), jnp.int32(0))
        _, _, d_loc, a_glob = lax.fori_loop(0, ndig_seg, scan_body, init)
        d_glob = wid * ndig_seg + d_loc

        mine = jnp.logical_and(suf < rank, rank <= suf + seg_sum)
        resv = jnp.where(lane == 0, d_glob,
                         jnp.where(lane == 1, a_glob, 0)).astype(jnp.int32)
        tmp16[...] = resv

        @pl.when(mine)
        def _():
            pltpu.sync_copy(tmp16, res.at[chain_row])

    p_h = jnp.int32(0)
    p_t = jnp.int32(0)
    r_h = jnp.int32(_R_H)
    r_t = jnp.int32(_R_T)

    for p in range(3):
        width = (11, 10, 10)[p]

        zero_hist()

        if p == 0:
            def process(buf):
                @plsc.parallel_loop(0, _CH // 16, unroll=8)
                def _(i):
                    bits = buf[pl.ds(i * 16, 16)] & 0x7FFFFFFF
                    idx = (lax.shift_right_logical(bits, 16) & 0x7FF0) | lane
                    plsc.addupdate_scatter(hist, [idx], ones)
        elif p == 1:
            ph, pt = p_h, p_t

            def process(buf):
                @plsc.parallel_loop(0, _CH // 16, unroll=8)
                def _(i):
                    bits = buf[pl.ds(i * 16, 16)] & 0x7FFFFFFF
                    pref = lax.shift_right_logical(bits, 20)
                    idx = (lax.shift_right_logical(bits, 6) & 0x3FF0) | lane
                    plsc.addupdate_scatter(hist, [idx], ones,
                                           mask=pref == ph)
                    plsc.addupdate_scatter(hist, [idx | _HALF], ones,
                                           mask=pref == pt)
        else:
            ph, pt = p_h, p_t

            def process(buf):
                @plsc.parallel_loop(0, _CH // 16, unroll=8)
                def _(i):
                    bits = buf[pl.ds(i * 16, 16)] & 0x7FFFFFFF
                    pref = lax.shift_right_logical(bits, 10)
                    idx = (lax.shift_left(bits, 4) & 0x3FF0) | lane
                    plsc.addupdate_scatter(hist, [idx], ones,
                                           mask=pref == ph)
                    plsc.addupdate_scatter(hist, [idx | _HALF], ones,
                                           mask=pref == pt)

        stream(process)

        pltpu.sync_copy(hist, sa_all.at[wid])
        plsc.subcore_barrier()

        if p == 0:
            find_chain(0, tbl_h, r_h, 2048, 128, 0)
            find_chain(0, tbl_t, r_t, 2048, 128, 1)
        else:
            find_chain(0, tbl_h, r_h, 1024, 64, 0)
            find_chain(_HALF, tbl_t, r_t, 1024, 64, 1)
        plsc.subcore_barrier()

        pltpu.sync_copy(res, rbuf)
        vh = rbuf[0, pl.ds(0, 16)]
        vt = rbuf[1, pl.ds(0, 16)]
        d_h, a_h = vh[0], vh[1]
        d_t, a_t = vt[0], vt[1]
        p_h = lax.shift_left(p_h, width) | d_h
        p_t = lax.shift_left(p_t, width) | d_t
        r_h = r_h - a_h
        r_t = r_t - a_t

    vi = jnp.where(lane == 0, p_h,
                   jnp.where(lane == 1, p_t, 0)).astype(jnp.int32)
    outb[...] = plsc.bitcast(vi, jnp.float32)

    @pl.when(wid == 0)
    def _():
        pltpu.sync_copy(outb, out_hbm)


@functools.partial(
    pl.kernel,
    out_type=(jax.ShapeDtypeStruct((16,), jnp.float32),
              jax.ShapeDtypeStruct((16, 16), jnp.int32),
              jax.ShapeDtypeStruct((16, 16), jnp.int32),
              jax.ShapeDtypeStruct((2, 16), jnp.int32)),
    mesh=plsc.VectorSubcoreMesh(core_axis_name="c", subcore_axis_name="s",
                                num_cores=1),
    compiler_params=pltpu.CompilerParams(needs_layout_passes=False),
    scratch_types=[
        pltpu.VMEM((_CH,), jnp.int32),
        pltpu.VMEM((_CH,), jnp.int32),
        pltpu.VMEM((_HSIZE,), jnp.int32),
        pltpu.VMEM((16, 2048), jnp.int32),
        pltpu.VMEM((16, 16), jnp.int32),
        pltpu.VMEM((2, 16), jnp.int32),
        pltpu.VMEM((16,), jnp.int32),
        pltpu.VMEM((16,), jnp.float32),
        pltpu.SemaphoreType.DMA,
        pltpu.SemaphoreType.DMA,
        pltpu.SemaphoreType.DMA,
        pltpu.VMEM_SHARED((16, _HSIZE), jnp.int32),
    ],
)
def _select_kernel(w_hbm, out_hbm, tbl_h, tbl_t, res, *rest):
    _sc_select(w_hbm, out_hbm, tbl_h, tbl_t, res, *rest)


_ROWS_PER_BLK = 256
_SQRT_INV2T = math.sqrt(0.5 / _TEMP)


def _apply_body(sel_ref, w_ref, o_ref):
    wh = sel_ref[0]
    wt = sel_ref[1]
    t = 0.5 * (wh + wt)
    a = (t * t) * (0.5 / _TEMP)
    w = w_ref[...]
    u = w * _SQRT_INV2T
    m = 0.5 + 0.5 * jnp.tanh(u * u - a)
    o_ref[...] = m * w


def kernel(weight):
    w_flat = weight.reshape(-1)
    w_i32 = lax.bitcast_convert_type(w_flat, jnp.int32)
    sel, _, _, _ = _select_kernel(w_i32)

    out = pl.pallas_call(
        _apply_body,
        grid=(weight.shape[0] // _ROWS_PER_BLK,),
        in_specs=[
            pl.BlockSpec(memory_space=pltpu.SMEM),
            pl.BlockSpec((_ROWS_PER_BLK, weight.shape[1]),
                         lambda i: (i, 0)),
        ],
        out_specs=pl.BlockSpec((_ROWS_PER_BLK, weight.shape[1]),
                               lambda i: (i, 0)),
        out_shape=jax.ShapeDtypeStruct(weight.shape, weight.dtype),
    )(sel, weight)
    return out


# speedup vs baseline: 41.5098x; 1.0033x over previous
import functools
import math

import jax
import jax.numpy as jnp
from jax import lax
from jax.experimental import pallas as pl
from jax.experimental.pallas import tpu as pltpu
from jax.experimental.pallas import tpu_sc as plsc

_SPARSITY = 0.9
_TEMP = 1e-05

_N = 4096 * 4096
_LIM = max(0, int((1.0 - _SPARSITY) * _N) - 1)
_R_H = _LIM + 1
_R_T = _LIM + 2

_NTILES = 16
_PER_TILE = _N // _NTILES
_CH = 8192
_NCH = _PER_TILE // _CH

_HSIZE = 32768
_HALF = _HSIZE // 2


def _sc_select(w_hbm, out_hbm, tbl_h, tbl_t, res, buf0, buf1, hist, seg,
               tbuf, rbuf, tmp16, outb, sem0, sem1, sem_seg, sa_all):
    wid = lax.axis_index("s")
    lane = lax.iota(jnp.int32, 16)
    ones = jnp.ones((16,), jnp.int32)
    zeros16 = jnp.zeros((16,), jnp.int32)
    base0 = wid * _PER_TILE

    def zero_hist():
        @plsc.parallel_loop(0, _HSIZE // 16, unroll=8)
        def _(i):
            hist[pl.ds(i * 16, 16)] = zeros16

    def stream(process):
        pltpu.async_copy(w_hbm.at[pl.ds(base0, _CH)], buf0, sem0)

        def pair(j, _):
            c0 = 2 * j
            pltpu.async_copy(
                w_hbm.at[pl.ds(base0 + (c0 + 1) * _CH, _CH)], buf1, sem1)
            pltpu.make_async_copy(
                w_hbm.at[pl.ds(0, _CH)], buf0, sem0).wait()
            process(buf0)

            @pl.when(j < _NCH // 2 - 1)
            def _():
                pltpu.async_copy(
                    w_hbm.at[pl.ds(base0 + (c0 + 2) * _CH, _CH)], buf0, sem0)

            pltpu.make_async_copy(
                w_hbm.at[pl.ds(0, _CH)], buf1, sem1).wait()
            process(buf1)
            return 0

        lax.fori_loop(0, _NCH // 2, pair, 0)

    def extract(vec, pos):
        return jnp.sum(jnp.where(lane == pos, vec, 0))

    def find_chain(base, tbl, rank, seg_entries, ndig_seg, chain_row):
        copies = []
        for t in range(_NTILES):
            copies.append(pltpu.async_copy(
                sa_all.at[t, pl.ds(base + wid * seg_entries, seg_entries)],
                seg.at[t, pl.ds(0, seg_entries)], sem_seg))
        for c in copies:
            c.wait()

        def tot_body(k, acc):
            for t in range(_NTILES):
                acc = acc + seg[t, pl.ds(k * 16, 16)]
            return acc
        accv = lax.fori_loop(0, seg_entries // 16, tot_body, zeros16)
        seg_sum = jnp.sum(accv)

        tmp16[...] = jnp.broadcast_to(seg_sum, (16,)).astype(jnp.int32)
        pltpu.sync_copy(tmp16, tbl.at[pl.ds(wid * 16, 16)])
        plsc.subcore_barrier()
        pltpu.sync_copy(tbl, tbuf)
        suf = jnp.int32(0)
        for t in range(_NTILES):
            s_t = tbuf[pl.ds(t * 16, 16)][0]
            suf = suf + jnp.where(jnp.int32(t) > wid, s_t, jnp.int32(0))

        def scan_body(i, carry):
            cum, found, d_sel, a_sel = carry
            k = ndig_seg - 1 - i
            dv = zeros16
            for t in range(_NTILES):
                dv = dv + seg[t, pl.ds(k * 16, 16)]
            tot = jnp.sum(dv)
            new_cum = cum + tot
            crossed = jnp.logical_and(suf + new_cum >= rank, found == 0)
            d_sel = jnp.where(crossed, k, d_sel)
            a_sel = jnp.where(crossed, suf + cum, a_sel)
            found = jnp.where(crossed, 1, found)
            return (new_cum, found, d_sel, a_sel)

        init = (jnp.int32(0), jnp.int32(0), jnp.int32(0), jnp.int32(0))
        _, _, d_loc, a_glob = lax.fori_loop(0, ndig_seg, scan_body, init)
        d_glob = wid * ndig_seg + d_loc

        mine = jnp.logical_and(suf < rank, rank <= suf + seg_sum)
        resv = jnp.where(lane == 0, d_glob,
                         jnp.where(lane == 1, a_glob, 0)).astype(jnp.int32)
        tmp16[...] = resv

        @pl.when(mine)
        def _():
            pltpu.sync_copy(tmp16, res.at[pl.ds(chain_row * 16, 16)])

    p_h = jnp.int32(0)
    p_t = jnp.int32(0)
    r_h = jnp.int32(_R_H)
    r_t = jnp.int32(_R_T)

    for p in range(3):
        width = (11, 10, 10)[p]

        zero_hist()

        if p == 0:
            def process(buf):
                @plsc.parallel_loop(0, _CH // 16, unroll=8)
                def _(i):
                    bits = buf[pl.ds(i * 16, 16)] & 0x7FFFFFFF
                    idx = (lax.shift_right_logical(bits, 16) & 0x7FF0) | lane
                    plsc.addupdate_scatter(hist, [idx], ones)
        elif p == 1:
            ph, pt = p_h, p_t

            def process(buf):
                @plsc.parallel_loop(0, _CH // 16, unroll=8)
                def _(i):
                    bits = buf[pl.ds(i * 16, 16)] & 0x7FFFFFFF
                    pref = lax.shift_right_logical(bits, 20)
                    idx = (lax.shift_right_logical(bits, 6) & 0x3FF0) | lane
                    plsc.addupdate_scatter(hist, [idx], ones,
                                           mask=pref == ph)
                    plsc.addupdate_scatter(hist, [idx | _HALF], ones,
                                           mask=pref == pt)
        else:
            ph, pt = p_h, p_t

            def process(buf):
                @plsc.parallel_loop(0, _CH // 16, unroll=8)
                def _(i):
                    bits = buf[pl.ds(i * 16, 16)] & 0x7FFFFFFF
                    pref = lax.shift_right_logical(bits, 10)
                    idx = (lax.shift_left(bits, 4) & 0x3FF0) | lane
                    plsc.addupdate_scatter(hist, [idx], ones,
                                           mask=pref == ph)
                    plsc.addupdate_scatter(hist, [idx | _HALF], ones,
                                           mask=pref == pt)

        stream(process)

        pltpu.sync_copy(hist, sa_all.at[wid])
        plsc.subcore_barrier()

        if p == 0:
            find_chain(0, tbl_h, r_h, 2048, 128, 0)
            find_chain(0, tbl_t, r_t, 2048, 128, 1)
        else:
            find_chain(0, tbl_h, r_h, 1024, 64, 0)
            find_chain(_HALF, tbl_t, r_t, 1024, 64, 1)
        plsc.subcore_barrier()

        pltpu.sync_copy(res, rbuf)
        vh = rbuf[pl.ds(0, 16)]
        vt = rbuf[pl.ds(16, 16)]
        d_h, a_h = vh[0], vh[1]
        d_t, a_t = vt[0], vt[1]
        p_h = lax.shift_left(p_h, width) | d_h
        p_t = lax.shift_left(p_t, width) | d_t
        r_h = r_h - a_h
        r_t = r_t - a_t

    vi = jnp.where(lane == 0, p_h,
                   jnp.where(lane == 1, p_t, 0)).astype(jnp.int32)
    outb[...] = plsc.bitcast(vi, jnp.float32)

    @pl.when(wid == 0)
    def _():
        pltpu.sync_copy(outb, out_hbm)


@functools.partial(
    pl.kernel,
    out_type=(jax.ShapeDtypeStruct((16,), jnp.float32),
              jax.ShapeDtypeStruct((256,), jnp.int32),
              jax.ShapeDtypeStruct((256,), jnp.int32),
              jax.ShapeDtypeStruct((32,), jnp.int32)),
    mesh=plsc.VectorSubcoreMesh(core_axis_name="c", subcore_axis_name="s",
                                num_cores=1),
    compiler_params=pltpu.CompilerParams(needs_layout_passes=False,
                                         use_tc_tiling_on_sc=True),
    scratch_types=[
        pltpu.VMEM((_CH,), jnp.int32),
        pltpu.VMEM((_CH,), jnp.int32),
        pltpu.VMEM((_HSIZE,), jnp.int32),
        pltpu.VMEM((16, 2048), jnp.int32),
        pltpu.VMEM((256,), jnp.int32),
        pltpu.VMEM((32,), jnp.int32),
        pltpu.VMEM((16,), jnp.int32),
        pltpu.VMEM((16,), jnp.float32),
        pltpu.SemaphoreType.DMA,
        pltpu.SemaphoreType.DMA,
        pltpu.SemaphoreType.DMA,
        pltpu.VMEM_SHARED((16, _HSIZE), jnp.int32),
    ],
)
def _select_kernel(w_hbm, out_hbm, tbl_h, tbl_t, res, *rest):
    _sc_select(w_hbm, out_hbm, tbl_h, tbl_t, res, *rest)


_ROWS_PER_BLK = 256
_SQRT_INV2T = math.sqrt(0.5 / _TEMP)


def _apply_body(sel_ref, w_ref, o_ref):
    wh = sel_ref[0]
    wt = sel_ref[1]
    t = 0.5 * (wh + wt)
    a = (t * t) * (0.5 / _TEMP)
    w = w_ref[...]
    u = w * _SQRT_INV2T
    m = 0.5 + 0.5 * jnp.tanh(u * u - a)
    o_ref[...] = m * w


def kernel(weight):
    w_flat = weight.reshape(-1)
    w_i32 = lax.bitcast_convert_type(w_flat, jnp.int32)
    sel, _, _, _ = _select_kernel(w_i32)

    out = pl.pallas_call(
        _apply_body,
        grid=(weight.shape[0] // _ROWS_PER_BLK,),
        in_specs=[
            pl.BlockSpec(memory_space=pltpu.SMEM),
            pl.BlockSpec((_ROWS_PER_BLK, weight.shape[1]),
                         lambda i: (i, 0)),
        ],
        out_specs=pl.BlockSpec((_ROWS_PER_BLK, weight.shape[1]),
                               lambda i: (i, 0)),
        out_shape=jax.ShapeDtypeStruct(weight.shape, weight.dtype),
    )(sel, weight)
    return out


# speedup vs baseline: 45.6600x; 1.1000x over previous
import functools
import math

import jax
import jax.numpy as jnp
from jax import lax
from jax.experimental import pallas as pl
from jax.experimental.pallas import tpu as pltpu
from jax.experimental.pallas import tpu_sc as plsc

_SPARSITY = 0.9
_TEMP = 1e-05

_N = 4096 * 4096
_LIM = max(0, int((1.0 - _SPARSITY) * _N) - 1)
_R_H = _LIM + 1
_R_T = _LIM + 2

_NTILES = 16
_PER_TILE = _N // _NTILES
_CH = 8192
_NCH = _PER_TILE // _CH

_HSIZE = 32768
_HALF = _HSIZE // 2


def _sc_select(w_hbm, out_hbm, tbl_h, tbl_t, res, buf0, buf1, hist, seg,
               tbuf, rbuf, tmp16, outb, sem0, sem1, sem_seg, sa_all):
    wid = lax.axis_index("s")
    lane = lax.iota(jnp.int32, 16)
    ones = jnp.ones((16,), jnp.int32)
    zeros16 = jnp.zeros((16,), jnp.int32)
    row0 = wid * (_PER_TILE // 4096)

    def zero_hist():
        @plsc.parallel_loop(0, _HSIZE // 16, unroll=8)
        def _(i):
            hist[pl.ds(i * 16, 16)] = zeros16

    _RCH = _CH // 4096

    def stream(process):
        pltpu.async_copy(w_hbm.at[pl.ds(row0, _RCH), :], buf0, sem0)

        def pair(j, _):
            c0 = 2 * j
            pltpu.async_copy(
                w_hbm.at[pl.ds(row0 + (c0 + 1) * _RCH, _RCH), :], buf1, sem1)
            pltpu.make_async_copy(
                w_hbm.at[pl.ds(0, _RCH), :], buf0, sem0).wait()
            process(buf0)

            @pl.when(j < _NCH // 2 - 1)
            def _():
                pltpu.async_copy(
                    w_hbm.at[pl.ds(row0 + (c0 + 2) * _RCH, _RCH), :],
                    buf0, sem0)

            pltpu.make_async_copy(
                w_hbm.at[pl.ds(0, _RCH), :], buf1, sem1).wait()
            process(buf1)
            return 0

        lax.fori_loop(0, _NCH // 2, pair, 0)

    def extract(vec, pos):
        return jnp.sum(jnp.where(lane == pos, vec, 0))

    def find_chain(base, tbl, rank, seg_entries, ndig_seg, chain_row):
        copies = []
        for t in range(_NTILES):
            copies.append(pltpu.async_copy(
                sa_all.at[t, pl.ds(base + wid * seg_entries, seg_entries)],
                seg.at[t, pl.ds(0, seg_entries)], sem_seg))
        for c in copies:
            c.wait()

        def tot_body(k, acc):
            for t in range(_NTILES):
                acc = acc + seg[t, pl.ds(k * 16, 16)]
            return acc
        accv = lax.fori_loop(0, seg_entries // 16, tot_body, zeros16)
        seg_sum = jnp.sum(accv)

        tmp16[...] = jnp.broadcast_to(seg_sum, (16,)).astype(jnp.int32)
        pltpu.sync_copy(tmp16, tbl.at[pl.ds(wid * 16, 16)])
        plsc.subcore_barrier()
        pltpu.sync_copy(tbl, tbuf)
        suf = jnp.int32(0)
        for t in range(_NTILES):
            s_t = tbuf[pl.ds(t * 16, 16)][0]
            suf = suf + jnp.where(jnp.int32(t) > wid, s_t, jnp.int32(0))

        def scan_body(i, carry):
            cum, found, d_sel, a_sel = carry
            k = ndig_seg - 1 - i
            dv = zeros16
            for t in range(_NTILES):
                dv = dv + seg[t, pl.ds(k * 16, 16)]
            tot = jnp.sum(dv)
            new_cum = cum + tot
            crossed = jnp.logical_and(suf + new_cum >= rank, found == 0)
            d_sel = jnp.where(crossed, k, d_sel)
            a_sel = jnp.where(crossed, suf + cum, a_sel)
            found = jnp.where(crossed, 1, found)
            return (new_cum, found, d_sel, a_sel)

        init = (jnp.int32(0), jnp.int32(0), jnp.int32(0), jnp.int32(0))
        _, _, d_loc, a_glob = lax.fori_loop(0, ndig_seg, scan_body, init)
        d_glob = wid * ndig_seg + d_loc

        mine = jnp.logical_and(suf < rank, rank <= suf + seg_sum)
        resv = jnp.where(lane == 0, d_glob,
                         jnp.where(lane == 1, a_glob, 0)).astype(jnp.int32)
        tmp16[...] = resv

        @pl.when(mine)
        def _():
            pltpu.sync_copy(tmp16, res.at[pl.ds(chain_row * 16, 16)])

    p_h = jnp.int32(0)
    p_t = jnp.int32(0)
    r_h = jnp.int32(_R_H)
    r_t = jnp.int32(_R_T)

    for p in range(3):
        width = (11, 10, 10)[p]

        zero_hist()

        if p == 0:
            def process(buf):
                for r in range(_RCH):
                    @plsc.parallel_loop(0, 4096 // 16, unroll=8)
                    def _(i, buf=buf, r=r):
                        bits = buf[r, pl.ds(i * 16, 16)] & 0x7FFFFFFF
                        idx = ((lax.shift_right_logical(bits, 16) & 0x7FF0)
                               | lane)
                        plsc.addupdate_scatter(hist, [idx], ones)
        elif p == 1:
            ph, pt = p_h, p_t

            def process(buf):
                for r in range(_RCH):
                    @plsc.parallel_loop(0, 4096 // 16, unroll=8)
                    def _(i, buf=buf, r=r):
                        bits = buf[r, pl.ds(i * 16, 16)] & 0x7FFFFFFF
                        pref = lax.shift_right_logical(bits, 20)
                        idx = ((lax.shift_right_logical(bits, 6) & 0x3FF0)
                               | lane)
                        plsc.addupdate_scatter(hist, [idx], ones,
                                               mask=pref == ph)
                        plsc.addupdate_scatter(hist, [idx | _HALF], ones,
                                               mask=pref == pt)
        else:
            ph, pt = p_h, p_t

            def process(buf):
                for r in range(_RCH):
                    @plsc.parallel_loop(0, 4096 // 16, unroll=8)
                    def _(i, buf=buf, r=r):
                        bits = buf[r, pl.ds(i * 16, 16)] & 0x7FFFFFFF
                        pref = lax.shift_right_logical(bits, 10)
                        idx = (lax.shift_left(bits, 4) & 0x3FF0) | lane
                        plsc.addupdate_scatter(hist, [idx], ones,
                                               mask=pref == ph)
                        plsc.addupdate_scatter(hist, [idx | _HALF], ones,
                                               mask=pref == pt)

        stream(process)

        pltpu.sync_copy(hist, sa_all.at[wid])
        plsc.subcore_barrier()

        if p == 0:
            find_chain(0, tbl_h, r_h, 2048, 128, 0)
            find_chain(0, tbl_t, r_t, 2048, 128, 1)
        else:
            find_chain(0, tbl_h, r_h, 1024, 64, 0)
            find_chain(_HALF, tbl_t, r_t, 1024, 64, 1)
        plsc.subcore_barrier()

        pltpu.sync_copy(res, rbuf)
        vh = rbuf[pl.ds(0, 16)]
        vt = rbuf[pl.ds(16, 16)]
        d_h, a_h = vh[0], vh[1]
        d_t, a_t = vt[0], vt[1]
        p_h = lax.shift_left(p_h, width) | d_h
        p_t = lax.shift_left(p_t, width) | d_t
        r_h = r_h - a_h
        r_t = r_t - a_t

    vi = jnp.where(lane == 0, p_h,
                   jnp.where(lane == 1, p_t, 0)).astype(jnp.int32)
    outb[...] = plsc.bitcast(vi, jnp.float32)

    @pl.when(wid == 0)
    def _():
        pltpu.sync_copy(outb, out_hbm)


@functools.partial(
    pl.kernel,
    out_type=(jax.ShapeDtypeStruct((16,), jnp.float32),
              jax.ShapeDtypeStruct((256,), jnp.int32),
              jax.ShapeDtypeStruct((256,), jnp.int32),
              jax.ShapeDtypeStruct((32,), jnp.int32)),
    mesh=plsc.VectorSubcoreMesh(core_axis_name="c", subcore_axis_name="s",
                                num_cores=1),
    compiler_params=pltpu.CompilerParams(needs_layout_passes=False,
                                         use_tc_tiling_on_sc=True),
    scratch_types=[
        pltpu.VMEM((_CH // 4096, 4096), jnp.int32),
        pltpu.VMEM((_CH // 4096, 4096), jnp.int32),
        pltpu.VMEM((_HSIZE,), jnp.int32),
        pltpu.VMEM((16, 2048), jnp.int32),
        pltpu.VMEM((256,), jnp.int32),
        pltpu.VMEM((32,), jnp.int32),
        pltpu.VMEM((16,), jnp.int32),
        pltpu.VMEM((16,), jnp.float32),
        pltpu.SemaphoreType.DMA,
        pltpu.SemaphoreType.DMA,
        pltpu.SemaphoreType.DMA,
        pltpu.VMEM_SHARED((16, _HSIZE), jnp.int32),
    ],
)
def _select_kernel(w_hbm, out_hbm, tbl_h, tbl_t, res, *rest):
    _sc_select(w_hbm, out_hbm, tbl_h, tbl_t, res, *rest)


_ROWS_PER_BLK = 256
_SQRT_INV2T = math.sqrt(0.5 / _TEMP)


def _apply_body(sel_ref, w_ref, o_ref):
    wh = sel_ref[0]
    wt = sel_ref[1]
    t = 0.5 * (wh + wt)
    a = (t * t) * (0.5 / _TEMP)
    w = w_ref[...]
    u = w * _SQRT_INV2T
    m = 0.5 + 0.5 * jnp.tanh(u * u - a)
    o_ref[...] = m * w


def kernel(weight):
    w_i32 = lax.bitcast_convert_type(weight, jnp.int32)
    sel, _, _, _ = _select_kernel(w_i32)

    out = pl.pallas_call(
        _apply_body,
        grid=(weight.shape[0] // _ROWS_PER_BLK,),
        in_specs=[
            pl.BlockSpec(memory_space=pltpu.SMEM),
            pl.BlockSpec((_ROWS_PER_BLK, weight.shape[1]),
                         lambda i: (i, 0)),
        ],
        out_specs=pl.BlockSpec((_ROWS_PER_BLK, weight.shape[1]),
                               lambda i: (i, 0)),
        out_shape=jax.ShapeDtypeStruct(weight.shape, weight.dtype),
    )(sel, weight)
    return out
